# Initial kernel scaffold; baseline (speedup 1.0000x reference)
import jax
import jax.numpy as jnp
from jax.experimental import pallas as pl


def kernel(atomic_numbers, atomic_properties, table, W1, b1, W2, b2):
    raise NotImplementedError("write your pallas kernel here")


# speedup vs baseline: 3.1042x; 3.1042x over previous
import jax
import jax.numpy as jnp
from jax.experimental import pallas as pl
from jax.experimental.pallas import tpu as pltpu

B, A, P, V, D = 4096, 200, 8, 92, 64
N = B * A
R = 2048
assert N % R == 0
G = N // R


def _body(idx_ref, prop_ref, table_ref, w1_ref, b1_ref, w2_ref, b2_ref, out_ref):
    idx = idx_ref[...]
    lanes = jax.lax.broadcasted_iota(jnp.int32, (R, 128), 1)
    onehot = (idx == lanes).astype(jnp.float32)
    elem = jnp.dot(onehot, table_ref[...], preferred_element_type=jnp.float32)
    h = jnp.dot(prop_ref[...], w1_ref[...], preferred_element_type=jnp.float32)
    h = jnp.maximum(h + b1_ref[...], 0.0)
    prop = jnp.dot(h, w2_ref[...], preferred_element_type=jnp.float32)
    out_ref[...] = elem + prop + b2_ref[...]


def kernel(atomic_numbers, atomic_properties, table, W1, b1, W2, b2):
    idx = (atomic_numbers.astype(jnp.int32) - 1).reshape(N, 1)
    props = atomic_properties.reshape(N, P)
    table_pad = jnp.zeros((128, D), jnp.float32).at[:V].set(table)
    out = pl.pallas_call(
        _body,
        grid=(G,),
        in_specs=[
            pl.BlockSpec((R, 1), lambda i: (i, 0)),
            pl.BlockSpec((R, P), lambda i: (i, 0)),
            pl.BlockSpec((128, D), lambda i: (0, 0)),
            pl.BlockSpec((P, 32), lambda i: (0, 0)),
            pl.BlockSpec((1, 32), lambda i: (0, 0)),
            pl.BlockSpec((32, D), lambda i: (0, 0)),
            pl.BlockSpec((1, D), lambda i: (0, 0)),
        ],
        out_specs=pl.BlockSpec((R, D), lambda i: (i, 0)),
        out_shape=jax.ShapeDtypeStruct((N, D), jnp.float32),
        compiler_params=pltpu.CompilerParams(
            dimension_semantics=("arbitrary",),
        ),
    )(idx, props, table_pad, W1, b1.reshape(1, 32), W2, b2.reshape(1, D))
    return out.reshape(B, A, D)
